# baseline (speedup 1.0000x reference)
import jax
import jax.numpy as jnp
from jax.experimental import pallas as pl
from jax.experimental.pallas import tpu as pltpu

_S = 4
_K = 1024
_D = 256
_N = 16384
_BLK = 512
_GRID = _N // _BLK
_PDR = _K // _GRID


def _rvq_body(x_ref, w_ref, q_ref, idx_ref, cnt_ref, loss_ref, comp_ref,
              w2_ref, wsq_ref):
    i = pl.program_id(0)

    @pl.when(i == 0)
    def _init():
        cnt_ref[...] = jnp.zeros_like(cnt_ref)
        loss_ref[...] = jnp.zeros_like(loss_ref)
        comp_ref[...] = jnp.zeros_like(comp_ref)
        for s in range(_S):
            ws = w_ref[s]
            w2_ref[s] = ws + ws
            wsq_ref[s] = jnp.sum(ws * ws, axis=1)

    x = x_ref[...]
    r = x
    qsum = jnp.zeros_like(x)
    idx_rows = []
    cnt_rows = []
    loss_rows = []
    comp_rows = []
    lane = jax.lax.broadcasted_iota(jnp.int32, (_BLK, _K), 1)
    for s in range(_S):
        w = w_ref[s]
        w2 = w2_ref[s]
        wsq = wsq_ref[s]
        rsq = jnp.sum(r * r, axis=1, keepdims=True)
        mm2 = jax.lax.dot_general(r, w2, (((1,), (1,)), ((), ())),
                                  preferred_element_type=jnp.float32)
        dist = (rsq + wsq[None, :]) - mm2
        dmin = jnp.min(dist, axis=1, keepdims=True)
        idx = jnp.min(jnp.where(dist == dmin, lane, _K), axis=1)
        onehot = (lane == idx[:, None]).astype(jnp.float32)
        q = jax.lax.dot_general(onehot, w, (((1,), (0,)), ((), ())),
                                preferred_element_type=jnp.float32)
        loss_rows.append(jnp.broadcast_to(jnp.sum(dmin), (128,)))
        cnt_rows.append(jnp.sum(onehot, axis=0))
        idx_rows.append(idx)
        qst = r + (q - r)
        qsum = qsum + qst
        r = x - qsum

        rows = w_ref[s, pl.ds(i * _PDR, _PDR), :]
        g2 = jax.lax.dot_general(rows, w2, (((1,), (1,)), ((), ())),
                                 preferred_element_type=jnp.float32)
        rsq_rows = jnp.sum(rows * rows, axis=1, keepdims=True)
        d2 = (rsq_rows + wsq[None, :]) - g2
        d = jnp.sqrt(jnp.maximum(d2, 1e-12))
        col = jax.lax.broadcasted_iota(jnp.int32, (_PDR, _K), 1)
        grow = i * _PDR + jax.lax.broadcasted_iota(jnp.int32, (_PDR, _K), 0)
        d = jnp.where(col > grow, d, 0.0)
        comp_rows.append(jnp.broadcast_to(jnp.sum(d), (128,)))

    q_ref[...] = qsum
    idx_ref[...] = jnp.stack(idx_rows)[None]
    cnt_ref[...] += jnp.stack(cnt_rows)
    loss_ref[...] += jnp.stack(loss_rows)
    comp_ref[...] += jnp.stack(comp_rows)


def kernel(x, W):
    quantized, idx_raw, cnt, loss, comp = pl.pallas_call(
        _rvq_body,
        grid=(_GRID,),
        in_specs=[
            pl.BlockSpec((_BLK, _D), lambda i: (i, 0)),
            pl.BlockSpec((_S, _K, _D), lambda i: (0, 0, 0)),
        ],
        out_specs=[
            pl.BlockSpec((_BLK, _D), lambda i: (i, 0)),
            pl.BlockSpec((1, _S, _BLK), lambda i: (i, 0, 0)),
            pl.BlockSpec((_S, _K), lambda i: (0, 0)),
            pl.BlockSpec((_S, 128), lambda i: (0, 0)),
            pl.BlockSpec((_S, 128), lambda i: (0, 0)),
        ],
        out_shape=[
            jax.ShapeDtypeStruct((_N, _D), jnp.float32),
            jax.ShapeDtypeStruct((_GRID, _S, _BLK), jnp.int32),
            jax.ShapeDtypeStruct((_S, _K), jnp.float32),
            jax.ShapeDtypeStruct((_S, 128), jnp.float32),
            jax.ShapeDtypeStruct((_S, 128), jnp.float32),
        ],
        scratch_shapes=[
            pltpu.VMEM((_S, _K, _D), jnp.float32),
            pltpu.VMEM((_S, _K), jnp.float32),
        ],
        compiler_params=pltpu.CompilerParams(
            dimension_semantics=("arbitrary",)),
    )(x, W)

    nd = float(_N * _D)
    npairs = _K * (_K - 1) // 2
    total_quant = jnp.float32(0.0)
    total_util = jnp.float32(0.0)
    total_compact = jnp.float32(0.0)
    for s in range(_S):
        m = loss[s, 0] / nd
        total_quant = total_quant + (m + 0.25 * m)
        total_util = total_util + jnp.mean(jnp.abs(cnt[s] - float(_N) / _K))
        total_compact = total_compact + 2.0 * (comp[s, 0] / npairs)
    indices = idx_raw.transpose(1, 0, 2).reshape(_S, _N)
    return quantized, total_quant, total_util, total_compact, indices


# speedup vs baseline: 1.1583x; 1.1583x over previous
import jax
import jax.numpy as jnp
from jax.experimental import pallas as pl
from jax.experimental.pallas import tpu as pltpu

_S = 4
_K = 1024
_D = 256
_N = 16384
_BLK = 1024
_GRID = _N // _BLK
_PDR = _K // _GRID


def _rvq_body(x_ref, w_ref, q_ref, idx_ref, cnt_ref, loss_ref, comp_ref):
    i = pl.program_id(0)

    @pl.when(i == 0)
    def _init():
        cnt_ref[...] = jnp.zeros_like(cnt_ref)
        loss_ref[...] = jnp.zeros_like(loss_ref)
        comp_ref[...] = jnp.zeros_like(comp_ref)

    x = x_ref[...]
    r = x
    qsum = jnp.zeros_like(x)
    idx_rows = []
    cnt_rows = []
    loss_rows = []
    comp_rows = []
    lane = jax.lax.broadcasted_iota(jnp.int32, (_BLK, _K), 1)
    for s in range(_S):
        w = w_ref[s]
        w2 = w + w
        wsq = jnp.sum(w * w, axis=1)
        rsq = jnp.sum(r * r, axis=1, keepdims=True)
        mm2 = jax.lax.dot_general(r, w2, (((1,), (1,)), ((), ())),
                                  preferred_element_type=jnp.float32)
        dist = (rsq + wsq[None, :]) - mm2
        dmin = jnp.min(dist, axis=1, keepdims=True)
        idx = jnp.min(jnp.where(dist == dmin, lane, _K), axis=1)
        onehot = (lane == idx[:, None]).astype(jnp.float32)
        q = jax.lax.dot_general(onehot, w, (((1,), (0,)), ((), ())),
                                preferred_element_type=jnp.float32)
        loss_rows.append(jnp.broadcast_to(jnp.sum(dmin), (128,)))
        cnt_rows.append(jnp.sum(onehot, axis=0))
        idx_rows.append(idx)
        qst = r + (q - r)
        qsum = qsum + qst
        r = x - qsum

        rows = w_ref[s, pl.ds(i * _PDR, _PDR), :]
        g2 = jax.lax.dot_general(rows, w2, (((1,), (1,)), ((), ())),
                                 preferred_element_type=jnp.float32)
        rsq_rows = jnp.sum(rows * rows, axis=1, keepdims=True)
        d2 = (rsq_rows + wsq[None, :]) - g2
        d = jnp.sqrt(jnp.maximum(d2, 1e-12))
        col = jax.lax.broadcasted_iota(jnp.int32, (_PDR, _K), 1)
        grow = i * _PDR + jax.lax.broadcasted_iota(jnp.int32, (_PDR, _K), 0)
        d = jnp.where(col > grow, d, 0.0)
        comp_rows.append(jnp.broadcast_to(jnp.sum(d), (128,)))

    q_ref[...] = qsum
    idx_ref[...] = jnp.stack(idx_rows)[None]
    cnt_ref[...] += jnp.stack(cnt_rows)
    loss_ref[...] += jnp.stack(loss_rows)
    comp_ref[...] += jnp.stack(comp_rows)


def kernel(x, W):
    quantized, idx_raw, cnt, loss, comp = pl.pallas_call(
        _rvq_body,
        grid=(_GRID,),
        in_specs=[
            pl.BlockSpec((_BLK, _D), lambda i: (i, 0)),
            pl.BlockSpec((_S, _K, _D), lambda i: (0, 0, 0)),
        ],
        out_specs=[
            pl.BlockSpec((_BLK, _D), lambda i: (i, 0)),
            pl.BlockSpec((1, _S, _BLK), lambda i: (i, 0, 0)),
            pl.BlockSpec((_S, _K), lambda i: (0, 0)),
            pl.BlockSpec((_S, 128), lambda i: (0, 0)),
            pl.BlockSpec((_S, 128), lambda i: (0, 0)),
        ],
        out_shape=[
            jax.ShapeDtypeStruct((_N, _D), jnp.float32),
            jax.ShapeDtypeStruct((_GRID, _S, _BLK), jnp.int32),
            jax.ShapeDtypeStruct((_S, _K), jnp.float32),
            jax.ShapeDtypeStruct((_S, 128), jnp.float32),
            jax.ShapeDtypeStruct((_S, 128), jnp.float32),
        ],
        compiler_params=pltpu.CompilerParams(
            dimension_semantics=("arbitrary",)),
    )(x, W)

    nd = float(_N * _D)
    npairs = _K * (_K - 1) // 2
    total_quant = jnp.float32(0.0)
    total_util = jnp.float32(0.0)
    total_compact = jnp.float32(0.0)
    for s in range(_S):
        m = loss[s, 0] / nd
        total_quant = total_quant + (m + 0.25 * m)
        total_util = total_util + jnp.mean(jnp.abs(cnt[s] - float(_N) / _K))
        total_compact = total_compact + 2.0 * (comp[s, 0] / npairs)
    indices = idx_raw.transpose(1, 0, 2).reshape(_S, _N)
    return quantized, total_quant, total_util, total_compact, indices


# speedup vs baseline: 1.1977x; 1.0340x over previous
import jax
import jax.numpy as jnp
from jax.experimental import pallas as pl
from jax.experimental.pallas import tpu as pltpu

_S = 4
_K = 1024
_D = 256
_N = 16384
_BLK = 2048
_GRID = _N // _BLK
_PDR = _K // _GRID


def _rvq_body(x_ref, w_ref, q_ref, idx_ref, cnt_ref, loss_ref, comp_ref):
    i = pl.program_id(0)

    @pl.when(i == 0)
    def _init():
        cnt_ref[...] = jnp.zeros_like(cnt_ref)
        loss_ref[...] = jnp.zeros_like(loss_ref)
        comp_ref[...] = jnp.zeros_like(comp_ref)

    x = x_ref[...]
    r = x
    qsum = jnp.zeros_like(x)
    idx_rows = []
    cnt_rows = []
    loss_rows = []
    comp_rows = []
    lane = jax.lax.broadcasted_iota(jnp.int32, (_BLK, _K), 1)
    for s in range(_S):
        w = w_ref[s]
        w2 = w + w
        wsq = jnp.sum(w * w, axis=1)
        rsq = jnp.sum(r * r, axis=1, keepdims=True)
        mm2 = jax.lax.dot_general(r, w2, (((1,), (1,)), ((), ())),
                                  preferred_element_type=jnp.float32)
        dist = (rsq + wsq[None, :]) - mm2
        dmin = jnp.min(dist, axis=1, keepdims=True)
        idx = jnp.min(jnp.where(dist == dmin, lane, _K), axis=1)
        onehot = (lane == idx[:, None]).astype(jnp.float32)
        q = jax.lax.dot_general(onehot, w, (((1,), (0,)), ((), ())),
                                preferred_element_type=jnp.float32)
        loss_rows.append(jnp.broadcast_to(jnp.sum(dmin), (128,)))
        cnt_rows.append(jnp.sum(onehot, axis=0))
        idx_rows.append(idx)
        qst = r + (q - r)
        qsum = qsum + qst
        r = x - qsum

        rows = w_ref[s, pl.ds(i * _PDR, _PDR), :]
        g2 = jax.lax.dot_general(rows, w2, (((1,), (1,)), ((), ())),
                                 preferred_element_type=jnp.float32)
        rsq_rows = jnp.sum(rows * rows, axis=1, keepdims=True)
        d2 = (rsq_rows + wsq[None, :]) - g2
        d = jnp.sqrt(jnp.maximum(d2, 1e-12))
        col = jax.lax.broadcasted_iota(jnp.int32, (_PDR, _K), 1)
        grow = i * _PDR + jax.lax.broadcasted_iota(jnp.int32, (_PDR, _K), 0)
        d = jnp.where(col > grow, d, 0.0)
        comp_rows.append(jnp.broadcast_to(jnp.sum(d), (128,)))

    q_ref[...] = qsum
    idx_ref[...] = jnp.stack(idx_rows)[None]
    cnt_ref[...] += jnp.stack(cnt_rows)
    loss_ref[...] += jnp.stack(loss_rows)
    comp_ref[...] += jnp.stack(comp_rows)


def kernel(x, W):
    quantized, idx_raw, cnt, loss, comp = pl.pallas_call(
        _rvq_body,
        grid=(_GRID,),
        in_specs=[
            pl.BlockSpec((_BLK, _D), lambda i: (i, 0)),
            pl.BlockSpec((_S, _K, _D), lambda i: (0, 0, 0)),
        ],
        out_specs=[
            pl.BlockSpec((_BLK, _D), lambda i: (i, 0)),
            pl.BlockSpec((1, _S, _BLK), lambda i: (i, 0, 0)),
            pl.BlockSpec((_S, _K), lambda i: (0, 0)),
            pl.BlockSpec((_S, 128), lambda i: (0, 0)),
            pl.BlockSpec((_S, 128), lambda i: (0, 0)),
        ],
        out_shape=[
            jax.ShapeDtypeStruct((_N, _D), jnp.float32),
            jax.ShapeDtypeStruct((_GRID, _S, _BLK), jnp.int32),
            jax.ShapeDtypeStruct((_S, _K), jnp.float32),
            jax.ShapeDtypeStruct((_S, 128), jnp.float32),
            jax.ShapeDtypeStruct((_S, 128), jnp.float32),
        ],
        compiler_params=pltpu.CompilerParams(
            dimension_semantics=("arbitrary",)),
    )(x, W)

    nd = float(_N * _D)
    npairs = _K * (_K - 1) // 2
    total_quant = jnp.float32(0.0)
    total_util = jnp.float32(0.0)
    total_compact = jnp.float32(0.0)
    for s in range(_S):
        m = loss[s, 0] / nd
        total_quant = total_quant + (m + 0.25 * m)
        total_util = total_util + jnp.mean(jnp.abs(cnt[s] - float(_N) / _K))
        total_compact = total_compact + 2.0 * (comp[s, 0] / npairs)
    indices = idx_raw.transpose(1, 0, 2).reshape(_S, _N)
    return quantized, total_quant, total_util, total_compact, indices


# speedup vs baseline: 1.3174x; 1.0999x over previous
import jax
import jax.numpy as jnp
from jax.experimental import pallas as pl
from jax.experimental.pallas import tpu as pltpu

_S = 4
_K = 1024
_D = 256
_N = 16384
_BLK = 2048
_GRID = _N // _BLK
_PDR = _K // _GRID


def _rvq_body(x_ref, w_ref, q_ref, idx_ref, cnt_ref, loss_ref, comp_ref):
    i = pl.program_id(0)

    @pl.when(i == 0)
    def _init():
        cnt_ref[...] = jnp.zeros_like(cnt_ref)
        loss_ref[...] = jnp.zeros_like(loss_ref)
        comp_ref[...] = jnp.zeros_like(comp_ref)

    x = x_ref[...]
    r = x
    qsum = jnp.zeros_like(x)
    idx_rows = []
    cnt_rows = []
    loss_rows = []
    comp_rows = []
    lane = jax.lax.broadcasted_iota(jnp.int32, (_BLK, _K), 1)
    for s in range(_S):
        w = w_ref[s]
        w2 = w + w
        wsq = jnp.sum(w * w, axis=1)
        rsq = jnp.sum(r * r, axis=1, keepdims=True)
        mm2 = jax.lax.dot_general(r, w2, (((1,), (1,)), ((), ())),
                                  preferred_element_type=jnp.float32)
        dist = (rsq + wsq[None, :]) - mm2
        dmin = jnp.min(dist, axis=1, keepdims=True)
        idx = jnp.min(jnp.where(dist == dmin, lane, _K), axis=1)
        onehot = (lane == idx[:, None]).astype(jnp.float32)
        q = jax.lax.dot_general(onehot, w, (((1,), (0,)), ((), ())),
                                preferred_element_type=jnp.float32)
        loss_rows.append(jnp.broadcast_to(jnp.sum(dmin), (128,)))
        ones_row = jnp.ones((8, _BLK), jnp.float32)
        cnt_rows.append(jax.lax.dot_general(
            ones_row, onehot, (((1,), (0,)), ((), ())),
            preferred_element_type=jnp.float32)[0])
        idx_rows.append(idx)
        qst = r + (q - r)
        qsum = qsum + qst
        r = x - qsum

        rows = w_ref[s, pl.ds(i * _PDR, _PDR), :]
        g2 = jax.lax.dot_general(rows, w2, (((1,), (1,)), ((), ())),
                                 preferred_element_type=jnp.float32)
        rsq_rows = jnp.sum(rows * rows, axis=1, keepdims=True)
        d2 = (rsq_rows + wsq[None, :]) - g2
        d = jnp.sqrt(jnp.maximum(d2, 1e-12))
        col = jax.lax.broadcasted_iota(jnp.int32, (_PDR, _K), 1)
        grow = i * _PDR + jax.lax.broadcasted_iota(jnp.int32, (_PDR, _K), 0)
        d = jnp.where(col > grow, d, 0.0)
        comp_rows.append(jnp.broadcast_to(jnp.sum(d), (128,)))

    q_ref[...] = qsum
    idx_ref[...] = jnp.stack(idx_rows)[None]
    cnt_ref[...] += jnp.stack(cnt_rows)
    loss_ref[...] += jnp.stack(loss_rows)
    comp_ref[...] += jnp.stack(comp_rows)


def kernel(x, W):
    quantized, idx_raw, cnt, loss, comp = pl.pallas_call(
        _rvq_body,
        grid=(_GRID,),
        in_specs=[
            pl.BlockSpec((_BLK, _D), lambda i: (i, 0)),
            pl.BlockSpec((_S, _K, _D), lambda i: (0, 0, 0)),
        ],
        out_specs=[
            pl.BlockSpec((_BLK, _D), lambda i: (i, 0)),
            pl.BlockSpec((1, _S, _BLK), lambda i: (i, 0, 0)),
            pl.BlockSpec((_S, _K), lambda i: (0, 0)),
            pl.BlockSpec((_S, 128), lambda i: (0, 0)),
            pl.BlockSpec((_S, 128), lambda i: (0, 0)),
        ],
        out_shape=[
            jax.ShapeDtypeStruct((_N, _D), jnp.float32),
            jax.ShapeDtypeStruct((_GRID, _S, _BLK), jnp.int32),
            jax.ShapeDtypeStruct((_S, _K), jnp.float32),
            jax.ShapeDtypeStruct((_S, 128), jnp.float32),
            jax.ShapeDtypeStruct((_S, 128), jnp.float32),
        ],
        compiler_params=pltpu.CompilerParams(
            dimension_semantics=("arbitrary",)),
    )(x, W)

    nd = float(_N * _D)
    npairs = _K * (_K - 1) // 2
    total_quant = jnp.float32(0.0)
    total_util = jnp.float32(0.0)
    total_compact = jnp.float32(0.0)
    for s in range(_S):
        m = loss[s, 0] / nd
        total_quant = total_quant + (m + 0.25 * m)
        total_util = total_util + jnp.mean(jnp.abs(cnt[s] - float(_N) / _K))
        total_compact = total_compact + 2.0 * (comp[s, 0] / npairs)
    indices = idx_raw.transpose(1, 0, 2).reshape(_S, _N)
    return quantized, total_quant, total_util, total_compact, indices
